# Initial kernel scaffold; baseline (speedup 1.0000x reference)
import jax
import jax.numpy as jnp
from jax.experimental import pallas as pl


def kernel(x1, x2, table):
    raise NotImplementedError("write your pallas kernel here")


# speedup vs baseline: 2.5626x; 2.5626x over previous
import functools

import jax
import jax.numpy as jnp
from jax import lax
from jax.experimental import pallas as pl
from jax.experimental.pallas import tpu as pltpu
from jax.experimental.pallas import tpu_sc as plsc

VOCAB = 1000000
D = 32
B = 16384
NF = 26
FT = 2 * NF
L = 16

NC = 2
NS = 16
NW = NC * NS
BPW = B // NW

CHUNK = 2
ROWS = CHUNK * FT
NCH = BPW // CHUNK
GRPS = CHUNK * 2
INV = 1.0 / NF

_mesh = plsc.VectorSubcoreMesh(core_axis_name="c", subcore_axis_name="s")


@functools.partial(
    pl.kernel,
    mesh=_mesh,
    compiler_params=pltpu.CompilerParams(use_tc_tiling_on_sc=False),
    out_type=jax.ShapeDtypeStruct((B * 2, D), jnp.float32),
    scratch_types=[
        pltpu.VMEM((NCH, ROWS), jnp.int32),
        pltpu.VMEM((ROWS, D), jnp.float32),
        pltpu.VMEM((ROWS, D), jnp.float32),
        pltpu.VMEM((BPW * 2, D), jnp.float32),
        pltpu.SemaphoreType.DMA,
        pltpu.SemaphoreType.DMA,
    ],
)
def _emb_pool(idx_hbm, table_hbm, out_hbm, idx_v, rows0, rows1, out_v,
              sem0, sem1):
    wid = lax.axis_index("s") * NC + lax.axis_index("c")
    base = wid * NCH

    pltpu.sync_copy(idx_hbm.at[pl.ds(base, NCH)], idx_v)

    bufs = (rows0, rows1)
    sems = (sem0, sem1)

    def gather(g, buf, sem):
        return pltpu.make_async_copy(table_hbm.at[idx_v.at[g]], buf, sem)

    gather(0, rows0, sem0).start()
    gather(1, rows1, sem1).start()

    def reduce_chunk(g, buf):
        for grp in range(GRPS):
            s = grp * NF
            a0 = buf[s, 0:L]
            a1 = buf[s, L:D]
            for j in range(1, NF):
                a0 = a0 + buf[s + j, 0:L]
                a1 = a1 + buf[s + j, L:D]
            orow = g * GRPS + grp
            out_v[orow, 0:L] = a0 * INV
            out_v[orow, L:D] = a1 * INV

    def body(i, carry):
        for b in range(2):
            g = 2 * i + b
            gather(g, bufs[b], sems[b]).wait()
            reduce_chunk(g, bufs[b])

            @pl.when(g < NCH - 2)
            def _():
                gather(g + 2, bufs[b], sems[b]).start()

        return carry

    lax.fori_loop(0, NCH // 2, body, 0)

    pltpu.sync_copy(out_v, out_hbm.at[pl.ds(base * GRPS, BPW * 2)])


def kernel(x1, x2, table):
    idx = jnp.concatenate(
        [x1.astype(jnp.int32), x2.astype(jnp.int32)], axis=1)
    idx = idx.reshape(B // CHUNK, ROWS)
    out = _emb_pool(idx, table)
    return out.reshape(B, 2, D)
